# baseline (speedup 1.0000x reference)
import functools

import jax
import jax.numpy as jnp
from jax import lax
from jax.experimental import pallas as pl
from jax.experimental.pallas import tpu as pltpu
from jax.experimental.pallas import tpu_sc as plsc

N = 10000
E = 320000
D = 128
DEPTH = 5
G = 128
VOCAB = 100
MAXDEPTH = 20

NPAD = 10240
ROWS_PER_SUB = NPAD // 16
NW = 32
K = 120
CH = 85
EPAD = NW * CH * K


NBUF = 3


def _sc_agg_body(h_hbm, src_hbm, dst_hbm, out_hbm, isrc, idst, *rest):
    c = lax.axis_index("c")
    s = lax.axis_index("s")
    w = s * 2 + c
    bufs = list(rest[:NBUF])
    acc = rest[NBUF]
    gsems = list(rest[NBUF + 1:2 * NBUF + 1])
    ssems = list(rest[2 * NBUF + 1:3 * NBUF + 1])
    dsems = list(rest[3 * NBUF + 1:4 * NBUF + 1])
    scsems = list(rest[4 * NBUF + 1:5 * NBUF + 1])

    def src_start(j, t):
        pltpu.async_copy(src_hbm.at[w, j], isrc.at[t], ssems[t])

    def src_wait(j, t):
        pltpu.make_async_copy(src_hbm.at[w, j], isrc.at[t], ssems[t]).wait()

    def dst_start(j, t):
        pltpu.async_copy(dst_hbm.at[w, j], idst.at[t], dsems[t])

    def dst_wait(j, t):
        pltpu.make_async_copy(dst_hbm.at[w, j], idst.at[t], dsems[t]).wait()

    def gather_start(b):
        pltpu.async_copy(h_hbm.at[isrc.at[b]], bufs[b], gsems[b])

    def gather_wait(b):
        pltpu.make_async_copy(h_hbm.at[isrc.at[b]], bufs[b], gsems[b]).wait()

    def scatter_start(b):
        pltpu.async_copy(bufs[b], acc.at[idst.at[b]], scsems[b], add=True)

    def scatter_wait(b):
        pltpu.make_async_copy(bufs[b], acc.at[idst.at[b]], scsems[b]).wait()

    for t in range(NBUF):
        src_start(t, t)
        dst_start(t, t)
    pltpu.sync_copy(h_hbm.at[pl.ds(s * ROWS_PER_SUB, ROWS_PER_SUB)],
                    acc.at[pl.ds(s * ROWS_PER_SUB, ROWS_PER_SUB)])
    plsc.subcore_barrier()
    for t in range(NBUF):
        src_wait(t, t)
        gather_start(t)

    def chunk_step(j, b, first, last):
        gather_wait(b)
        dst_wait(j, b)
        scatter_start(b)
        prev = (b + NBUF - 1) % NBUF
        if not first:
            scatter_wait(prev)

            @pl.when(j + NBUF - 1 < CH)
            def _():
                src_start(j + NBUF - 1, prev)
                dst_start(j + NBUF - 1, prev)
                src_wait(j + NBUF - 1, prev)
                gather_start(prev)

    chunk_step(0, 0, True, False)

    def round_body(r, carry):
        for b in range(NBUF):
            chunk_step(NBUF * r + b + 1, (b + 1) % NBUF, False, False)
        return carry

    lax.fori_loop(0, (CH - 1) // NBUF, round_body, 0)
    scatter_wait((CH - 1) % NBUF)

    plsc.subcore_barrier()
    pltpu.sync_copy(acc.at[pl.ds(s * ROWS_PER_SUB, ROWS_PER_SUB)],
                    out_hbm.at[c, pl.ds(s * ROWS_PER_SUB, ROWS_PER_SUB)])


def _sc_agg(h, src3, dst3):
    mesh = plsc.VectorSubcoreMesh(core_axis_name="c", subcore_axis_name="s",
                                  num_cores=2, num_subcores=16)
    kern = pl.kernel(
        _sc_agg_body,
        out_type=jax.ShapeDtypeStruct((2, NPAD, D), jnp.float32),
        mesh=mesh,
        scratch_types=[
            pltpu.VMEM((NBUF, K), jnp.int32),
            pltpu.VMEM((NBUF, K), jnp.int32),
        ] + [pltpu.VMEM((K, D), jnp.float32)] * NBUF
          + [pltpu.VMEM_SHARED((NPAD, D), jnp.float32)]
          + [pltpu.SemaphoreType.DMA] * (4 * NBUF),
    )
    return kern(h, src3, dst3)


R = 1024
NB = NPAD // R

_HI = lax.Precision.HIGHEST


def _dot(a, b, prec=None):
    return jnp.dot(a, b, preferred_element_type=jnp.float32, precision=prec)


def _seg_sum(P, v):
    return lax.dot_general(P, v, (((0,), (0,)), ((), ())),
                           preferred_element_type=jnp.float32, precision=_HI)


def _onehot(b_ref):
    cols = lax.broadcasted_iota(jnp.int32, (1, G), 1)
    return (b_ref[...] == cols).astype(jnp.float32)


def _row_spec():
    return pl.BlockSpec((R, None) if False else (R, D), lambda i: (i, 0))


_BS_ROWS = pl.BlockSpec((R, D), lambda i: (i, 0))
_BS_COL1 = pl.BlockSpec((R, 1), lambda i: (i, 0))
_BS_ACC_GD = pl.BlockSpec((G, D), lambda i: (0, 0))
_BS_ACC_1G = pl.BlockSpec((1, G), lambda i: (0, 0))
_BS_ACC_G1 = pl.BlockSpec((G, 1), lambda i: (0, 0))


def _full(shape):
    return pl.BlockSpec(shape, lambda i: tuple(0 for _ in shape))


def _embed_body(x_ref, dep_ref, emb_ref, demb_ref, h_out):
    cols = lax.broadcasted_iota(jnp.int32, (1, 128), 1)
    px = (x_ref[...] == cols).astype(jnp.float32)
    pd = (dep_ref[...] == cols).astype(jnp.float32)
    h_out[...] = _dot(px, emb_ref[...], _HI) + _dot(pd, demb_ref[...], _HI)


def _embed(xp, dp, embp, depthp):
    return pl.pallas_call(
        _embed_body,
        grid=(NB,),
        in_specs=[_BS_COL1, _BS_COL1, _full((128, D)), _full((128, D))],
        out_specs=_BS_ROWS,
        out_shape=jax.ShapeDtypeStruct((NPAD, D), jnp.float32),
    )(xp, dp, embp, depthp)


def _mlp_body(h_ref, a_ref, b_ref, W1r, b1r, W2r, b2r, z_out, segz_out, cnt_out):
    h = h_ref[...]
    z = a_ref[0] + a_ref[1] - h
    z = jnp.maximum(_dot(z, W1r[...]) + b1r[...], 0.0)
    z = _dot(z, W2r[...]) + b2r[...]
    z_out[...] = z
    P = _onehot(b_ref)

    @pl.when(pl.program_id(0) == 0)
    def _():
        segz_out[...] = jnp.zeros_like(segz_out)
        cnt_out[...] = jnp.zeros_like(cnt_out)

    segz_out[...] += _seg_sum(P, z)
    cnt_out[...] += jnp.sum(P, axis=0, keepdims=True)


def _mlp(h, acc, bp, W1, b1, W2, b2):
    return pl.pallas_call(
        _mlp_body,
        grid=(NB,),
        in_specs=[_BS_ROWS, pl.BlockSpec((2, R, D), lambda i: (0, i, 0)),
                  _BS_COL1, _full((D, D)), _full((1, D)), _full((D, D)),
                  _full((1, D))],
        out_specs=[_BS_ROWS, _BS_ACC_GD, _BS_ACC_1G],
        out_shape=[jax.ShapeDtypeStruct((NPAD, D), jnp.float32),
                   jax.ShapeDtypeStruct((G, D), jnp.float32),
                   jax.ShapeDtypeStruct((1, G), jnp.float32)],
    )(h, acc, bp, W1, b1.reshape(1, D), W2, b2.reshape(1, D))


def _center_body(z_ref, b_ref, segz_ref, cnt_ref, gsr, out_out, segv_out):
    cnt = jnp.maximum(cnt_ref[...], 1.0).reshape(G, 1)
    mean = segz_ref[...] / cnt
    P = _onehot(b_ref)
    out = z_ref[...] - gsr[...] * _dot(P, mean, _HI)
    out_out[...] = out

    @pl.when(pl.program_id(0) == 0)
    def _():
        segv_out[...] = jnp.zeros_like(segv_out)

    segv_out[...] += _seg_sum(P, out * out)


def _center(z, bp, segz, cnt, gs):
    return pl.pallas_call(
        _center_body,
        grid=(NB,),
        in_specs=[_BS_ROWS, _BS_COL1, _full((G, D)), _full((1, G)),
                  _full((1, D))],
        out_specs=[_BS_ROWS, _BS_ACC_GD],
        out_shape=[jax.ShapeDtypeStruct((NPAD, D), jnp.float32),
                   jax.ShapeDtypeStruct((G, D), jnp.float32)],
    )(z, bp, segz, cnt, gs.reshape(1, D))


def _gate_body(last, out_ref, b_ref, segv_ref, cnt_ref, gwr, gbr, G1r, gb1r,
               G2r, gb2r, h_out, gate_out, mg_out):
    cnt = jnp.maximum(cnt_ref[...], 1.0).reshape(G, 1)
    inv = 1.0 / jnp.sqrt(segv_ref[...] / cnt + 1e-5)
    P = _onehot(b_ref)
    z = gwr[...] * out_ref[...] * _dot(P, inv, _HI) + gbr[...]
    if not last:
        z = jnp.maximum(z, 0.0)
    rows = lax.broadcasted_iota(jnp.int32, (R, 1), 0) + pl.program_id(0) * R
    z = jnp.where(rows < N, z, 0.0)
    h_out[...] = z
    gm = jnp.maximum(_dot(z, G1r[...]) + gb1r[...], 0.0)
    gate = _dot(gm, G2r[...]) + gb2r[...]
    gate_out[...] = gate

    @pl.when(pl.program_id(0) == 0)
    def _():
        mg_out[...] = jnp.full_like(mg_out, -jnp.inf)

    blockmax = jnp.max(jnp.where(P > 0, gate, -jnp.inf), axis=0, keepdims=True)
    mg_out[...] = jnp.maximum(mg_out[...], blockmax)


def _gate(last, out, bp, segv, cnt, gw, gb, G1, gb1, G2, gb2):
    return pl.pallas_call(
        functools.partial(_gate_body, last),
        grid=(NB,),
        in_specs=[_BS_ROWS, _BS_COL1, _full((G, D)), _full((1, G)),
                  _full((1, D)), _full((1, D)), _full((D, 2 * D)),
                  _full((1, 2 * D)), _full((2 * D, 1)), _full((1, 1))],
        out_specs=[_BS_ROWS, _BS_COL1, _BS_ACC_1G],
        out_shape=[jax.ShapeDtypeStruct((NPAD, D), jnp.float32),
                   jax.ShapeDtypeStruct((NPAD, 1), jnp.float32),
                   jax.ShapeDtypeStruct((1, G), jnp.float32)],
    )(out, bp, segv, cnt, gw.reshape(1, D), gb.reshape(1, D), G1,
      gb1.reshape(1, 2 * D), G2, gb2.reshape(1, 1))


def _pool_body(h_ref, gate_ref, b_ref, mg_ref, ssum_out, u_out):
    mg = mg_ref[...]
    mg = jnp.where(jnp.isfinite(mg), mg, 0.0)
    P = _onehot(b_ref)
    e = jnp.exp(gate_ref[...] - _dot(P, mg.reshape(G, 1), _HI))
    rows = lax.broadcasted_iota(jnp.int32, (R, 1), 0) + pl.program_id(0) * R
    e = jnp.where(rows < N, e, 0.0)

    @pl.when(pl.program_id(0) == 0)
    def _():
        ssum_out[...] = jnp.zeros_like(ssum_out)
        u_out[...] = jnp.zeros_like(u_out)

    ssum_out[...] += _seg_sum(P, e)
    u_out[...] += _seg_sum(P, e * h_ref[...])


def _pool(h, gate, bp, mg):
    return pl.pallas_call(
        _pool_body,
        grid=(NB,),
        in_specs=[_BS_ROWS, _BS_COL1, _BS_COL1, _full((1, G))],
        out_specs=[_BS_ACC_G1, _BS_ACC_GD],
        out_shape=[jax.ShapeDtypeStruct((G, 1), jnp.float32),
                   jax.ShapeDtypeStruct((G, D), jnp.float32)],
    )(h, gate, bp, mg)


def _final_body(u_ref, s_ref, out_ref):
    total = jnp.zeros((G, D), jnp.float32)
    for d in range(DEPTH):
        hg = u_ref[d] / jnp.maximum(s_ref[d], 1e-16)
        out_ref[d] = hg
        total = total + hg
    out_ref[DEPTH] = total * (1.0 / DEPTH)


def _final(us, ss):
    return pl.pallas_call(
        _final_body,
        out_shape=jax.ShapeDtypeStruct((DEPTH + 1, G, D), jnp.float32),
    )(us, ss)


def _layer(d, h, acc, bp, W1, b1, W2, b2, gw, gb, gs, G1, gb1, G2, gb2):
    last = d == DEPTH - 1
    z, segz, cnt = _mlp(h, acc, bp, W1, b1, W2, b2)
    out, segv = _center(z, bp, segz, cnt, gs)
    h_new, gate, mg = _gate(last, out, bp, segv, cnt, gw, gb, G1, gb1, G2, gb2)
    ssum, u = _pool(h_new, gate, bp, mg)
    return h_new, ssum, u


def kernel(x, edge_index, node_depth, batch, node_emb, depth_emb, W1, b1, W2, b2,
           gn_weight, gn_bias, gn_mean_scale, G1, gb1, G2, gb2):
    src = edge_index[0].astype(jnp.int32)
    dst = edge_index[1].astype(jnp.int32)
    src3 = jnp.concatenate([src, jnp.zeros((EPAD - E,), jnp.int32)]).reshape(NW, CH, K)
    dst3 = jnp.concatenate([dst, jnp.full((EPAD - E,), N, jnp.int32)]).reshape(NW, CH, K)
    xp = jnp.concatenate([x.astype(jnp.int32),
                          jnp.full((NPAD - N,), VOCAB, jnp.int32)]).reshape(NPAD, 1)
    dp = jnp.concatenate([node_depth.astype(jnp.int32),
                          jnp.full((NPAD - N,), MAXDEPTH, jnp.int32)]).reshape(NPAD, 1)
    bp = jnp.concatenate([batch.astype(jnp.int32),
                          jnp.full((NPAD - N,), G, jnp.int32)]).reshape(NPAD, 1)
    embp = jnp.zeros((128, D), jnp.float32).at[:VOCAB].set(node_emb)
    depthp = jnp.zeros((128, D), jnp.float32).at[:MAXDEPTH].set(depth_emb)

    h = _embed(xp, dp, embp, depthp)
    us, ss = [], []
    for d in range(DEPTH):
        acc = _sc_agg(h, src3, dst3)
        h, ssum, u = _layer(d, h, acc, bp, W1[d], b1[d], W2[d], b2[d],
                            gn_weight[d], gn_bias[d], gn_mean_scale[d],
                            G1[d], gb1[d], G2[d], gb2[d])
        us.append(u)
        ss.append(ssum)
    return _final(jnp.stack(us, axis=0), jnp.stack(ss, axis=0))


# speedup vs baseline: 1.5896x; 1.5896x over previous
import functools

import jax
import jax.numpy as jnp
from jax import lax
from jax.experimental import pallas as pl
from jax.experimental.pallas import tpu as pltpu
from jax.experimental.pallas import tpu_sc as plsc

N = 10000
E = 320000
D = 128
DEPTH = 5
G = 128
VOCAB = 100
MAXDEPTH = 20

NPAD = 10240
ROWS_PER_SUB = NPAD // 16
NW = 32
K = 120
CH = 84
EPAD = NW * CH * K


NBUF = 3


def _sc_agg_body(h_hbm, src_hbm, dst_hbm, out_hbm, isrc, idst, buf0, buf1,
                 buf2, acc, gs0, gs1, gs2, ss0, ss1, ss2, ds0, ds1, ds2):
    c = lax.axis_index("c")
    s = lax.axis_index("s")
    w = s * 2 + c
    bufs = [buf0, buf1, buf2]
    gsems = [gs0, gs1, gs2]
    ssems = [ss0, ss1, ss2]
    dsems = [ds0, ds1, ds2]

    def src_start(j, t):
        pltpu.async_copy(src_hbm.at[w, j], isrc.at[t], ssems[t])

    def src_wait(j, t):
        pltpu.make_async_copy(src_hbm.at[w, j], isrc.at[t], ssems[t]).wait()

    def dst_start(j, t):
        pltpu.async_copy(dst_hbm.at[w, j], idst.at[t], dsems[t])

    def dst_wait(j, t):
        pltpu.make_async_copy(dst_hbm.at[w, j], idst.at[t], dsems[t]).wait()

    def gather_start(b):
        pltpu.async_copy(h_hbm.at[isrc.at[b]], bufs[b], gsems[b])

    def gather_wait(b):
        pltpu.make_async_copy(h_hbm.at[isrc.at[b]], bufs[b], gsems[b]).wait()

    for t in range(NBUF):
        src_start(t, t)
        dst_start(t, t)
    pltpu.sync_copy(h_hbm.at[pl.ds(s * ROWS_PER_SUB, ROWS_PER_SUB)],
                    acc.at[pl.ds(s * ROWS_PER_SUB, ROWS_PER_SUB)])
    plsc.subcore_barrier()
    for t in range(NBUF):
        src_wait(t, t)
        gather_start(t)

    def chunk_step(j, b):
        gather_wait(b)
        dst_wait(j, b)
        pltpu.sync_copy(bufs[b], acc.at[idst.at[b]], add=True)

        @pl.when(j + NBUF < CH)
        def _():
            src_start(j + NBUF, b)
            dst_start(j + NBUF, b)
            src_wait(j + NBUF, b)
            gather_start(b)

    def round_body(r, carry):
        for b in range(NBUF):
            chunk_step(NBUF * r + b, b)
        return carry

    lax.fori_loop(0, CH // NBUF, round_body, 0)

    plsc.subcore_barrier()
    pltpu.sync_copy(acc.at[pl.ds(s * ROWS_PER_SUB, ROWS_PER_SUB)],
                    out_hbm.at[c, pl.ds(s * ROWS_PER_SUB, ROWS_PER_SUB)])


def _sc_agg(h, src3, dst3):
    mesh = plsc.VectorSubcoreMesh(core_axis_name="c", subcore_axis_name="s",
                                  num_cores=2, num_subcores=16)
    kern = pl.kernel(
        _sc_agg_body,
        out_type=jax.ShapeDtypeStruct((2, NPAD, D), jnp.float32),
        mesh=mesh,
        scratch_types=[
            pltpu.VMEM((NBUF, K), jnp.int32),
            pltpu.VMEM((NBUF, K), jnp.int32),
            pltpu.VMEM((K, D), jnp.float32),
            pltpu.VMEM((K, D), jnp.float32),
            pltpu.VMEM((K, D), jnp.float32),
        ] + [pltpu.VMEM_SHARED((NPAD, D), jnp.float32)]
          + [pltpu.SemaphoreType.DMA] * 9,
    )
    return kern(h, src3, dst3)


R = 1024
NB = NPAD // R

_HI = lax.Precision.HIGHEST


def _dot(a, b, prec=None):
    return jnp.dot(a, b, preferred_element_type=jnp.float32, precision=prec)


def _seg_sum(P, v):
    return lax.dot_general(P, v, (((0,), (0,)), ((), ())),
                           preferred_element_type=jnp.float32, precision=_HI)


def _onehot(b_ref):
    cols = lax.broadcasted_iota(jnp.int32, (1, G), 1)
    return (b_ref[...] == cols).astype(jnp.float32)


def _row_spec():
    return pl.BlockSpec((R, None) if False else (R, D), lambda i: (i, 0))


_BS_ROWS = pl.BlockSpec((R, D), lambda i: (i, 0))
_BS_COL1 = pl.BlockSpec((R, 1), lambda i: (i, 0))
_BS_ACC_GD = pl.BlockSpec((G, D), lambda i: (0, 0))
_BS_ACC_1G = pl.BlockSpec((1, G), lambda i: (0, 0))
_BS_ACC_G1 = pl.BlockSpec((G, 1), lambda i: (0, 0))


def _full(shape):
    return pl.BlockSpec(shape, lambda i: tuple(0 for _ in shape))


def _embed_body(x_ref, dep_ref, emb_ref, demb_ref, h_out):
    cols = lax.broadcasted_iota(jnp.int32, (1, 128), 1)
    px = (x_ref[...] == cols).astype(jnp.float32)
    pd = (dep_ref[...] == cols).astype(jnp.float32)
    h_out[...] = _dot(px, emb_ref[...], _HI) + _dot(pd, demb_ref[...], _HI)


def _embed(xp, dp, embp, depthp):
    return pl.pallas_call(
        _embed_body,
        grid=(NB,),
        in_specs=[_BS_COL1, _BS_COL1, _full((128, D)), _full((128, D))],
        out_specs=_BS_ROWS,
        out_shape=jax.ShapeDtypeStruct((NPAD, D), jnp.float32),
    )(xp, dp, embp, depthp)


def _mlp_body(h_ref, a_ref, b_ref, W1r, b1r, W2r, b2r, z_out, segz_out, cnt_out):
    h = h_ref[...]
    z = a_ref[0] + a_ref[1] - h
    z = jnp.maximum(_dot(z, W1r[...]) + b1r[...], 0.0)
    z = _dot(z, W2r[...]) + b2r[...]
    z_out[...] = z
    P = _onehot(b_ref)

    @pl.when(pl.program_id(0) == 0)
    def _():
        segz_out[...] = jnp.zeros_like(segz_out)
        cnt_out[...] = jnp.zeros_like(cnt_out)

    segz_out[...] += _seg_sum(P, z)
    cnt_out[...] += jnp.sum(P, axis=0, keepdims=True)


def _mlp(h, acc, bp, W1, b1, W2, b2):
    return pl.pallas_call(
        _mlp_body,
        grid=(NB,),
        in_specs=[_BS_ROWS, pl.BlockSpec((2, R, D), lambda i: (0, i, 0)),
                  _BS_COL1, _full((D, D)), _full((1, D)), _full((D, D)),
                  _full((1, D))],
        out_specs=[_BS_ROWS, _BS_ACC_GD, _BS_ACC_1G],
        out_shape=[jax.ShapeDtypeStruct((NPAD, D), jnp.float32),
                   jax.ShapeDtypeStruct((G, D), jnp.float32),
                   jax.ShapeDtypeStruct((1, G), jnp.float32)],
    )(h, acc, bp, W1, b1.reshape(1, D), W2, b2.reshape(1, D))


def _center_body(z_ref, b_ref, segz_ref, cnt_ref, gsr, out_out, segv_out):
    cnt = jnp.maximum(cnt_ref[...], 1.0).reshape(G, 1)
    mean = segz_ref[...] / cnt
    P = _onehot(b_ref)
    out = z_ref[...] - gsr[...] * _dot(P, mean, _HI)
    out_out[...] = out

    @pl.when(pl.program_id(0) == 0)
    def _():
        segv_out[...] = jnp.zeros_like(segv_out)

    segv_out[...] += _seg_sum(P, out * out)


def _center(z, bp, segz, cnt, gs):
    return pl.pallas_call(
        _center_body,
        grid=(NB,),
        in_specs=[_BS_ROWS, _BS_COL1, _full((G, D)), _full((1, G)),
                  _full((1, D))],
        out_specs=[_BS_ROWS, _BS_ACC_GD],
        out_shape=[jax.ShapeDtypeStruct((NPAD, D), jnp.float32),
                   jax.ShapeDtypeStruct((G, D), jnp.float32)],
    )(z, bp, segz, cnt, gs.reshape(1, D))


def _gate_body(last, out_ref, b_ref, segv_ref, cnt_ref, gwr, gbr, G1r, gb1r,
               G2r, gb2r, h_out, gate_out, mg_out):
    cnt = jnp.maximum(cnt_ref[...], 1.0).reshape(G, 1)
    inv = 1.0 / jnp.sqrt(segv_ref[...] / cnt + 1e-5)
    P = _onehot(b_ref)
    z = gwr[...] * out_ref[...] * _dot(P, inv, _HI) + gbr[...]
    if not last:
        z = jnp.maximum(z, 0.0)
    rows = lax.broadcasted_iota(jnp.int32, (R, 1), 0) + pl.program_id(0) * R
    z = jnp.where(rows < N, z, 0.0)
    h_out[...] = z
    gm = jnp.maximum(_dot(z, G1r[...]) + gb1r[...], 0.0)
    gate = _dot(gm, G2r[...]) + gb2r[...]
    gate_out[...] = gate

    @pl.when(pl.program_id(0) == 0)
    def _():
        mg_out[...] = jnp.full_like(mg_out, -jnp.inf)

    blockmax = jnp.max(jnp.where(P > 0, gate, -jnp.inf), axis=0, keepdims=True)
    mg_out[...] = jnp.maximum(mg_out[...], blockmax)


def _gate(last, out, bp, segv, cnt, gw, gb, G1, gb1, G2, gb2):
    return pl.pallas_call(
        functools.partial(_gate_body, last),
        grid=(NB,),
        in_specs=[_BS_ROWS, _BS_COL1, _full((G, D)), _full((1, G)),
                  _full((1, D)), _full((1, D)), _full((D, 2 * D)),
                  _full((1, 2 * D)), _full((2 * D, 1)), _full((1, 1))],
        out_specs=[_BS_ROWS, _BS_COL1, _BS_ACC_1G],
        out_shape=[jax.ShapeDtypeStruct((NPAD, D), jnp.float32),
                   jax.ShapeDtypeStruct((NPAD, 1), jnp.float32),
                   jax.ShapeDtypeStruct((1, G), jnp.float32)],
    )(out, bp, segv, cnt, gw.reshape(1, D), gb.reshape(1, D), G1,
      gb1.reshape(1, 2 * D), G2, gb2.reshape(1, 1))


def _pool_body(h_ref, gate_ref, b_ref, mg_ref, ssum_out, u_out):
    mg = mg_ref[...]
    mg = jnp.where(jnp.isfinite(mg), mg, 0.0)
    P = _onehot(b_ref)
    e = jnp.exp(gate_ref[...] - _dot(P, mg.reshape(G, 1), _HI))
    rows = lax.broadcasted_iota(jnp.int32, (R, 1), 0) + pl.program_id(0) * R
    e = jnp.where(rows < N, e, 0.0)

    @pl.when(pl.program_id(0) == 0)
    def _():
        ssum_out[...] = jnp.zeros_like(ssum_out)
        u_out[...] = jnp.zeros_like(u_out)

    ssum_out[...] += _seg_sum(P, e)
    u_out[...] += _seg_sum(P, e * h_ref[...])


def _pool(h, gate, bp, mg):
    return pl.pallas_call(
        _pool_body,
        grid=(NB,),
        in_specs=[_BS_ROWS, _BS_COL1, _BS_COL1, _full((1, G))],
        out_specs=[_BS_ACC_G1, _BS_ACC_GD],
        out_shape=[jax.ShapeDtypeStruct((G, 1), jnp.float32),
                   jax.ShapeDtypeStruct((G, D), jnp.float32)],
    )(h, gate, bp, mg)


def _final_body(u_ref, s_ref, out_ref):
    total = jnp.zeros((G, D), jnp.float32)
    for d in range(DEPTH):
        hg = u_ref[d] / jnp.maximum(s_ref[d], 1e-16)
        out_ref[d] = hg
        total = total + hg
    out_ref[DEPTH] = total * (1.0 / DEPTH)


def _final(us, ss):
    return pl.pallas_call(
        _final_body,
        out_shape=jax.ShapeDtypeStruct((DEPTH + 1, G, D), jnp.float32),
    )(us, ss)


def _layer(d, h, acc, bp, W1, b1, W2, b2, gw, gb, gs, G1, gb1, G2, gb2):
    last = d == DEPTH - 1
    z, segz, cnt = _mlp(h, acc, bp, W1, b1, W2, b2)
    out, segv = _center(z, bp, segz, cnt, gs)
    h_new, gate, mg = _gate(last, out, bp, segv, cnt, gw, gb, G1, gb1, G2, gb2)
    ssum, u = _pool(h_new, gate, bp, mg)
    return h_new, ssum, u


def kernel(x, edge_index, node_depth, batch, node_emb, depth_emb, W1, b1, W2, b2,
           gn_weight, gn_bias, gn_mean_scale, G1, gb1, G2, gb2):
    src = edge_index[0].astype(jnp.int32)
    dst = edge_index[1].astype(jnp.int32)
    src3 = jnp.concatenate([src, jnp.zeros((EPAD - E,), jnp.int32)]).reshape(NW, CH, K)
    dst3 = jnp.concatenate([dst, jnp.full((EPAD - E,), N, jnp.int32)]).reshape(NW, CH, K)
    xp = jnp.concatenate([x.astype(jnp.int32),
                          jnp.full((NPAD - N,), VOCAB, jnp.int32)]).reshape(NPAD, 1)
    dp = jnp.concatenate([node_depth.astype(jnp.int32),
                          jnp.full((NPAD - N,), MAXDEPTH, jnp.int32)]).reshape(NPAD, 1)
    bp = jnp.concatenate([batch.astype(jnp.int32),
                          jnp.full((NPAD - N,), G, jnp.int32)]).reshape(NPAD, 1)
    embp = jnp.zeros((128, D), jnp.float32).at[:VOCAB].set(node_emb)
    depthp = jnp.zeros((128, D), jnp.float32).at[:MAXDEPTH].set(depth_emb)

    h = _embed(xp, dp, embp, depthp)
    us, ss = [], []
    for d in range(DEPTH):
        acc = _sc_agg(h, src3, dst3)
        h, ssum, u = _layer(d, h, acc, bp, W1[d], b1[d], W2[d], b2[d],
                            gn_weight[d], gn_bias[d], gn_mean_scale[d],
                            G1[d], gb1[d], G2[d], gb2[d])
        us.append(u)
        ss.append(ssum)
    return _final(jnp.stack(us, axis=0), jnp.stack(ss, axis=0))


# speedup vs baseline: 1.5975x; 1.0050x over previous
import functools

import jax
import jax.numpy as jnp
from jax import lax
from jax.experimental import pallas as pl
from jax.experimental.pallas import tpu as pltpu
from jax.experimental.pallas import tpu_sc as plsc

N = 10000
E = 320000
D = 128
DEPTH = 5
G = 128
VOCAB = 100
MAXDEPTH = 20

NPAD = 10240
ROWS_PER_SUB = NPAD // 16
NW = 32
K = 120
CH = 84
EPAD = NW * CH * K


NBUF = 3


def _sc_agg_body(h_hbm, src_hbm, dst_hbm, out_hbm, isrc, idst, buf0, buf1,
                 buf2, acc, gs0, gs1, gs2, ss0, ss1, ss2, ds0, ds1, ds2):
    c = lax.axis_index("c")
    s = lax.axis_index("s")
    w = s * 2 + c
    bufs = [buf0, buf1, buf2]
    gsems = [gs0, gs1, gs2]
    ssems = [ss0, ss1, ss2]
    dsems = [ds0, ds1, ds2]

    def src_start(j, t):
        pltpu.async_copy(src_hbm.at[w, j], isrc.at[t], ssems[t])

    def src_wait(j, t):
        pltpu.make_async_copy(src_hbm.at[w, j], isrc.at[t], ssems[t]).wait()

    def dst_start(j, t):
        pltpu.async_copy(dst_hbm.at[w, j], idst.at[t], dsems[t])

    def dst_wait(j, t):
        pltpu.make_async_copy(dst_hbm.at[w, j], idst.at[t], dsems[t]).wait()

    def gather_start(b):
        pltpu.async_copy(h_hbm.at[isrc.at[b]], bufs[b], gsems[b])

    def gather_wait(b):
        pltpu.make_async_copy(h_hbm.at[isrc.at[b]], bufs[b], gsems[b]).wait()

    for t in range(NBUF):
        src_start(t, t)
        dst_start(t, t)
    for t in range(NBUF):
        src_wait(t, t)
        gather_start(t)
    pltpu.sync_copy(h_hbm.at[pl.ds(s * ROWS_PER_SUB, ROWS_PER_SUB)],
                    acc.at[pl.ds(s * ROWS_PER_SUB, ROWS_PER_SUB)])
    plsc.subcore_barrier()

    def chunk_step(j, b):
        gather_wait(b)
        dst_wait(j, b)
        pltpu.sync_copy(bufs[b], acc.at[idst.at[b]], add=True)

        @pl.when(j + NBUF < CH)
        def _():
            src_start(j + NBUF, b)
            dst_start(j + NBUF, b)
            src_wait(j + NBUF, b)
            gather_start(b)

    def round_body(r, carry):
        for b in range(NBUF):
            chunk_step(NBUF * r + b, b)
        return carry

    lax.fori_loop(0, CH // NBUF, round_body, 0)

    plsc.subcore_barrier()
    pltpu.sync_copy(acc.at[pl.ds(s * ROWS_PER_SUB, ROWS_PER_SUB)],
                    out_hbm.at[c, pl.ds(s * ROWS_PER_SUB, ROWS_PER_SUB)])


def _sc_agg(h, src3, dst3):
    mesh = plsc.VectorSubcoreMesh(core_axis_name="c", subcore_axis_name="s",
                                  num_cores=2, num_subcores=16)
    kern = pl.kernel(
        _sc_agg_body,
        out_type=jax.ShapeDtypeStruct((2, NPAD, D), jnp.float32),
        mesh=mesh,
        scratch_types=[
            pltpu.VMEM((NBUF, K), jnp.int32),
            pltpu.VMEM((NBUF, K), jnp.int32),
            pltpu.VMEM((K, D), jnp.float32),
            pltpu.VMEM((K, D), jnp.float32),
            pltpu.VMEM((K, D), jnp.float32),
        ] + [pltpu.VMEM_SHARED((NPAD, D), jnp.float32)]
          + [pltpu.SemaphoreType.DMA] * 9,
    )
    return kern(h, src3, dst3)


R = 1024
NB = NPAD // R

_HI = lax.Precision.HIGHEST


def _dot(a, b, prec=None):
    return jnp.dot(a, b, preferred_element_type=jnp.float32, precision=prec)


def _seg_sum(P, v):
    return lax.dot_general(P, v, (((0,), (0,)), ((), ())),
                           preferred_element_type=jnp.float32, precision=_HI)


def _onehot(b_ref):
    cols = lax.broadcasted_iota(jnp.int32, (1, G), 1)
    return (b_ref[...] == cols).astype(jnp.float32)


def _row_spec():
    return pl.BlockSpec((R, None) if False else (R, D), lambda i: (i, 0))


_BS_ROWS = pl.BlockSpec((R, D), lambda i: (i, 0))
_BS_COL1 = pl.BlockSpec((R, 1), lambda i: (i, 0))
_BS_ACC_GD = pl.BlockSpec((G, D), lambda i: (0, 0))
_BS_ACC_1G = pl.BlockSpec((1, G), lambda i: (0, 0))
_BS_ACC_G1 = pl.BlockSpec((G, 1), lambda i: (0, 0))


def _full(shape):
    return pl.BlockSpec(shape, lambda i: tuple(0 for _ in shape))


def _embed_body(x_ref, dep_ref, emb_ref, demb_ref, h_out):
    cols = lax.broadcasted_iota(jnp.int32, (1, 128), 1)
    px = (x_ref[...] == cols).astype(jnp.float32)
    pd = (dep_ref[...] == cols).astype(jnp.float32)
    h_out[...] = _dot(px, emb_ref[...], _HI) + _dot(pd, demb_ref[...], _HI)


def _embed(xp, dp, embp, depthp):
    return pl.pallas_call(
        _embed_body,
        grid=(NB,),
        in_specs=[_BS_COL1, _BS_COL1, _full((128, D)), _full((128, D))],
        out_specs=_BS_ROWS,
        out_shape=jax.ShapeDtypeStruct((NPAD, D), jnp.float32),
    )(xp, dp, embp, depthp)


def _mlp_body(h_ref, a_ref, b_ref, W1r, b1r, W2r, b2r, z_out, segz_out, cnt_out):
    h = h_ref[...]
    z = a_ref[0] + a_ref[1] - h
    z = jnp.maximum(_dot(z, W1r[...]) + b1r[...], 0.0)
    z = _dot(z, W2r[...]) + b2r[...]
    z_out[...] = z
    P = _onehot(b_ref)

    @pl.when(pl.program_id(0) == 0)
    def _():
        segz_out[...] = jnp.zeros_like(segz_out)
        cnt_out[...] = jnp.zeros_like(cnt_out)

    segz_out[...] += _seg_sum(P, z)
    cnt_out[...] += jnp.sum(P, axis=0, keepdims=True)


def _mlp(h, acc, bp, W1, b1, W2, b2):
    return pl.pallas_call(
        _mlp_body,
        grid=(NB,),
        in_specs=[_BS_ROWS, pl.BlockSpec((2, R, D), lambda i: (0, i, 0)),
                  _BS_COL1, _full((D, D)), _full((1, D)), _full((D, D)),
                  _full((1, D))],
        out_specs=[_BS_ROWS, _BS_ACC_GD, _BS_ACC_1G],
        out_shape=[jax.ShapeDtypeStruct((NPAD, D), jnp.float32),
                   jax.ShapeDtypeStruct((G, D), jnp.float32),
                   jax.ShapeDtypeStruct((1, G), jnp.float32)],
    )(h, acc, bp, W1, b1.reshape(1, D), W2, b2.reshape(1, D))


def _center_body(z_ref, b_ref, segz_ref, cnt_ref, gsr, out_out, segv_out):
    cnt = jnp.maximum(cnt_ref[...], 1.0).reshape(G, 1)
    mean = segz_ref[...] / cnt
    P = _onehot(b_ref)
    out = z_ref[...] - gsr[...] * _dot(P, mean, _HI)
    out_out[...] = out

    @pl.when(pl.program_id(0) == 0)
    def _():
        segv_out[...] = jnp.zeros_like(segv_out)

    segv_out[...] += _seg_sum(P, out * out)


def _center(z, bp, segz, cnt, gs):
    return pl.pallas_call(
        _center_body,
        grid=(NB,),
        in_specs=[_BS_ROWS, _BS_COL1, _full((G, D)), _full((1, G)),
                  _full((1, D))],
        out_specs=[_BS_ROWS, _BS_ACC_GD],
        out_shape=[jax.ShapeDtypeStruct((NPAD, D), jnp.float32),
                   jax.ShapeDtypeStruct((G, D), jnp.float32)],
    )(z, bp, segz, cnt, gs.reshape(1, D))


def _gate_body(last, out_ref, b_ref, segv_ref, cnt_ref, gwr, gbr, G1r, gb1r,
               G2r, gb2r, h_out, gate_out, mg_out):
    cnt = jnp.maximum(cnt_ref[...], 1.0).reshape(G, 1)
    inv = 1.0 / jnp.sqrt(segv_ref[...] / cnt + 1e-5)
    P = _onehot(b_ref)
    z = gwr[...] * out_ref[...] * _dot(P, inv, _HI) + gbr[...]
    if not last:
        z = jnp.maximum(z, 0.0)
    rows = lax.broadcasted_iota(jnp.int32, (R, 1), 0) + pl.program_id(0) * R
    z = jnp.where(rows < N, z, 0.0)
    h_out[...] = z
    gm = jnp.maximum(_dot(z, G1r[...]) + gb1r[...], 0.0)
    gate = _dot(gm, G2r[...]) + gb2r[...]
    gate_out[...] = gate

    @pl.when(pl.program_id(0) == 0)
    def _():
        mg_out[...] = jnp.full_like(mg_out, -jnp.inf)

    blockmax = jnp.max(jnp.where(P > 0, gate, -jnp.inf), axis=0, keepdims=True)
    mg_out[...] = jnp.maximum(mg_out[...], blockmax)


def _gate(last, out, bp, segv, cnt, gw, gb, G1, gb1, G2, gb2):
    return pl.pallas_call(
        functools.partial(_gate_body, last),
        grid=(NB,),
        in_specs=[_BS_ROWS, _BS_COL1, _full((G, D)), _full((1, G)),
                  _full((1, D)), _full((1, D)), _full((D, 2 * D)),
                  _full((1, 2 * D)), _full((2 * D, 1)), _full((1, 1))],
        out_specs=[_BS_ROWS, _BS_COL1, _BS_ACC_1G],
        out_shape=[jax.ShapeDtypeStruct((NPAD, D), jnp.float32),
                   jax.ShapeDtypeStruct((NPAD, 1), jnp.float32),
                   jax.ShapeDtypeStruct((1, G), jnp.float32)],
    )(out, bp, segv, cnt, gw.reshape(1, D), gb.reshape(1, D), G1,
      gb1.reshape(1, 2 * D), G2, gb2.reshape(1, 1))


def _pool_body(h_ref, gate_ref, b_ref, mg_ref, ssum_out, u_out):
    mg = mg_ref[...]
    mg = jnp.where(jnp.isfinite(mg), mg, 0.0)
    P = _onehot(b_ref)
    e = jnp.exp(gate_ref[...] - _dot(P, mg.reshape(G, 1), _HI))
    rows = lax.broadcasted_iota(jnp.int32, (R, 1), 0) + pl.program_id(0) * R
    e = jnp.where(rows < N, e, 0.0)

    @pl.when(pl.program_id(0) == 0)
    def _():
        ssum_out[...] = jnp.zeros_like(ssum_out)
        u_out[...] = jnp.zeros_like(u_out)

    ssum_out[...] += _seg_sum(P, e)
    u_out[...] += _seg_sum(P, e * h_ref[...])


def _pool(h, gate, bp, mg):
    return pl.pallas_call(
        _pool_body,
        grid=(NB,),
        in_specs=[_BS_ROWS, _BS_COL1, _BS_COL1, _full((1, G))],
        out_specs=[_BS_ACC_G1, _BS_ACC_GD],
        out_shape=[jax.ShapeDtypeStruct((G, 1), jnp.float32),
                   jax.ShapeDtypeStruct((G, D), jnp.float32)],
    )(h, gate, bp, mg)


def _final_body(u_ref, s_ref, out_ref):
    total = jnp.zeros((G, D), jnp.float32)
    for d in range(DEPTH):
        hg = u_ref[d] / jnp.maximum(s_ref[d], 1e-16)
        out_ref[d] = hg
        total = total + hg
    out_ref[DEPTH] = total * (1.0 / DEPTH)


def _final(us, ss):
    return pl.pallas_call(
        _final_body,
        out_shape=jax.ShapeDtypeStruct((DEPTH + 1, G, D), jnp.float32),
    )(us, ss)


def _layer(d, h, acc, bp, W1, b1, W2, b2, gw, gb, gs, G1, gb1, G2, gb2):
    last = d == DEPTH - 1
    z, segz, cnt = _mlp(h, acc, bp, W1, b1, W2, b2)
    out, segv = _center(z, bp, segz, cnt, gs)
    h_new, gate, mg = _gate(last, out, bp, segv, cnt, gw, gb, G1, gb1, G2, gb2)
    ssum, u = _pool(h_new, gate, bp, mg)
    return h_new, ssum, u


def kernel(x, edge_index, node_depth, batch, node_emb, depth_emb, W1, b1, W2, b2,
           gn_weight, gn_bias, gn_mean_scale, G1, gb1, G2, gb2):
    src = edge_index[0].astype(jnp.int32)
    dst = edge_index[1].astype(jnp.int32)
    src3 = jnp.concatenate([src, jnp.zeros((EPAD - E,), jnp.int32)]).reshape(NW, CH, K)
    dst3 = jnp.concatenate([dst, jnp.full((EPAD - E,), N, jnp.int32)]).reshape(NW, CH, K)
    xp = jnp.concatenate([x.astype(jnp.int32),
                          jnp.full((NPAD - N,), VOCAB, jnp.int32)]).reshape(NPAD, 1)
    dp = jnp.concatenate([node_depth.astype(jnp.int32),
                          jnp.full((NPAD - N,), MAXDEPTH, jnp.int32)]).reshape(NPAD, 1)
    bp = jnp.concatenate([batch.astype(jnp.int32),
                          jnp.full((NPAD - N,), G, jnp.int32)]).reshape(NPAD, 1)
    embp = jnp.zeros((128, D), jnp.float32).at[:VOCAB].set(node_emb)
    depthp = jnp.zeros((128, D), jnp.float32).at[:MAXDEPTH].set(depth_emb)

    h = _embed(xp, dp, embp, depthp)
    us, ss = [], []
    for d in range(DEPTH):
        acc = _sc_agg(h, src3, dst3)
        h, ssum, u = _layer(d, h, acc, bp, W1[d], b1[d], W2[d], b2[d],
                            gn_weight[d], gn_bias[d], gn_mean_scale[d],
                            G1[d], gb1[d], G2[d], gb2[d])
        us.append(u)
        ss.append(ssum)
    return _final(jnp.stack(us, axis=0), jnp.stack(ss, axis=0))


# speedup vs baseline: 1.5977x; 1.0001x over previous
import functools

import jax
import jax.numpy as jnp
from jax import lax
from jax.experimental import pallas as pl
from jax.experimental.pallas import tpu as pltpu
from jax.experimental.pallas import tpu_sc as plsc

N = 10000
E = 320000
D = 128
DEPTH = 5
G = 128
VOCAB = 100
MAXDEPTH = 20

NPAD = 10240
ROWS_PER_SUB = NPAD // 16
NW = 32
K = 120
CH = 84
EPAD = NW * CH * K


NBUF = 3


def _sc_agg_body(h_hbm, src_hbm, dst_hbm, out_hbm, isrc, idst, buf0, buf1,
                 buf2, acc, gs0, gs1, gs2, ss0, ss1, ss2, ds0, ds1, ds2,
                 g2s0, g2s1, g2s2):
    c = lax.axis_index("c")
    s = lax.axis_index("s")
    w = s * 2 + c
    bufs = [buf0, buf1, buf2]
    gsems = [gs0, gs1, gs2]
    ssems = [ss0, ss1, ss2]
    dsems = [ds0, ds1, ds2]
    g2sems = [g2s0, g2s1, g2s2]

    def src_start(j, t):
        pltpu.async_copy(src_hbm.at[w, j], isrc.at[t], ssems[t])

    def src_wait(j, t):
        pltpu.make_async_copy(src_hbm.at[w, j], isrc.at[t], ssems[t]).wait()

    def dst_start(j, t):
        pltpu.async_copy(dst_hbm.at[w, j], idst.at[t], dsems[t])

    def dst_wait(j, t):
        pltpu.make_async_copy(dst_hbm.at[w, j], idst.at[t], dsems[t]).wait()

    KA = 64

    def gather_start(b):
        pltpu.async_copy(h_hbm.at[isrc.at[b, pl.ds(0, KA)]],
                         bufs[b].at[pl.ds(0, KA)], gsems[b])
        pltpu.async_copy(h_hbm.at[isrc.at[b, pl.ds(KA, K - KA)]],
                         bufs[b].at[pl.ds(KA, K - KA)], g2sems[b])

    def gather_wait(b):
        pltpu.make_async_copy(h_hbm.at[isrc.at[b, pl.ds(0, KA)]],
                              bufs[b].at[pl.ds(0, KA)], gsems[b]).wait()
        pltpu.make_async_copy(h_hbm.at[isrc.at[b, pl.ds(KA, K - KA)]],
                              bufs[b].at[pl.ds(KA, K - KA)], g2sems[b]).wait()

    for t in range(NBUF):
        src_start(t, t)
        dst_start(t, t)
    for t in range(NBUF):
        src_wait(t, t)
        gather_start(t)
    pltpu.sync_copy(h_hbm.at[pl.ds(s * ROWS_PER_SUB, ROWS_PER_SUB)],
                    acc.at[pl.ds(s * ROWS_PER_SUB, ROWS_PER_SUB)])
    plsc.subcore_barrier()

    def chunk_step(j, b):
        gather_wait(b)
        dst_wait(j, b)
        pltpu.sync_copy(bufs[b], acc.at[idst.at[b]], add=True)

        @pl.when(j + NBUF < CH)
        def _():
            src_start(j + NBUF, b)
            dst_start(j + NBUF, b)
            src_wait(j + NBUF, b)
            gather_start(b)

    def round_body(r, carry):
        for b in range(NBUF):
            chunk_step(NBUF * r + b, b)
        return carry

    lax.fori_loop(0, CH // NBUF, round_body, 0)

    plsc.subcore_barrier()
    pltpu.sync_copy(acc.at[pl.ds(s * ROWS_PER_SUB, ROWS_PER_SUB)],
                    out_hbm.at[c, pl.ds(s * ROWS_PER_SUB, ROWS_PER_SUB)])


def _sc_agg(h, src3, dst3):
    mesh = plsc.VectorSubcoreMesh(core_axis_name="c", subcore_axis_name="s",
                                  num_cores=2, num_subcores=16)
    kern = pl.kernel(
        _sc_agg_body,
        out_type=jax.ShapeDtypeStruct((2, NPAD, D), jnp.float32),
        mesh=mesh,
        scratch_types=[
            pltpu.VMEM((NBUF, K), jnp.int32),
            pltpu.VMEM((NBUF, K), jnp.int32),
            pltpu.VMEM((K, D), jnp.float32),
            pltpu.VMEM((K, D), jnp.float32),
            pltpu.VMEM((K, D), jnp.float32),
        ] + [pltpu.VMEM_SHARED((NPAD, D), jnp.float32)]
          + [pltpu.SemaphoreType.DMA] * 12,
    )
    return kern(h, src3, dst3)


R = 1024
NB = NPAD // R

_HI = lax.Precision.HIGHEST


def _dot(a, b, prec=None):
    return jnp.dot(a, b, preferred_element_type=jnp.float32, precision=prec)


def _seg_sum(P, v):
    return lax.dot_general(P, v, (((0,), (0,)), ((), ())),
                           preferred_element_type=jnp.float32, precision=_HI)


def _onehot(b_ref):
    cols = lax.broadcasted_iota(jnp.int32, (1, G), 1)
    return (b_ref[...] == cols).astype(jnp.float32)


def _row_spec():
    return pl.BlockSpec((R, None) if False else (R, D), lambda i: (i, 0))


_BS_ROWS = pl.BlockSpec((R, D), lambda i: (i, 0))
_BS_COL1 = pl.BlockSpec((R, 1), lambda i: (i, 0))
_BS_ACC_GD = pl.BlockSpec((G, D), lambda i: (0, 0))
_BS_ACC_1G = pl.BlockSpec((1, G), lambda i: (0, 0))
_BS_ACC_G1 = pl.BlockSpec((G, 1), lambda i: (0, 0))


def _full(shape):
    return pl.BlockSpec(shape, lambda i: tuple(0 for _ in shape))


def _embed_body(x_ref, dep_ref, emb_ref, demb_ref, h_out):
    cols = lax.broadcasted_iota(jnp.int32, (1, 128), 1)
    px = (x_ref[...] == cols).astype(jnp.float32)
    pd = (dep_ref[...] == cols).astype(jnp.float32)
    h_out[...] = _dot(px, emb_ref[...], _HI) + _dot(pd, demb_ref[...], _HI)


def _embed(xp, dp, embp, depthp):
    return pl.pallas_call(
        _embed_body,
        grid=(NB,),
        in_specs=[_BS_COL1, _BS_COL1, _full((128, D)), _full((128, D))],
        out_specs=_BS_ROWS,
        out_shape=jax.ShapeDtypeStruct((NPAD, D), jnp.float32),
    )(xp, dp, embp, depthp)


def _mlp_body(h_ref, a_ref, b_ref, W1r, b1r, W2r, b2r, z_out, segz_out, cnt_out):
    h = h_ref[...]
    z = a_ref[0] + a_ref[1] - h
    z = jnp.maximum(_dot(z, W1r[...]) + b1r[...], 0.0)
    z = _dot(z, W2r[...]) + b2r[...]
    z_out[...] = z
    P = _onehot(b_ref)

    @pl.when(pl.program_id(0) == 0)
    def _():
        segz_out[...] = jnp.zeros_like(segz_out)
        cnt_out[...] = jnp.zeros_like(cnt_out)

    segz_out[...] += _seg_sum(P, z)
    cnt_out[...] += jnp.sum(P, axis=0, keepdims=True)


def _mlp(h, acc, bp, W1, b1, W2, b2):
    return pl.pallas_call(
        _mlp_body,
        grid=(NB,),
        in_specs=[_BS_ROWS, pl.BlockSpec((2, R, D), lambda i: (0, i, 0)),
                  _BS_COL1, _full((D, D)), _full((1, D)), _full((D, D)),
                  _full((1, D))],
        out_specs=[_BS_ROWS, _BS_ACC_GD, _BS_ACC_1G],
        out_shape=[jax.ShapeDtypeStruct((NPAD, D), jnp.float32),
                   jax.ShapeDtypeStruct((G, D), jnp.float32),
                   jax.ShapeDtypeStruct((1, G), jnp.float32)],
    )(h, acc, bp, W1, b1.reshape(1, D), W2, b2.reshape(1, D))


def _center_body(z_ref, b_ref, segz_ref, cnt_ref, gsr, out_out, segv_out):
    cnt = jnp.maximum(cnt_ref[...], 1.0).reshape(G, 1)
    mean = segz_ref[...] / cnt
    P = _onehot(b_ref)
    out = z_ref[...] - gsr[...] * _dot(P, mean, _HI)
    out_out[...] = out

    @pl.when(pl.program_id(0) == 0)
    def _():
        segv_out[...] = jnp.zeros_like(segv_out)

    segv_out[...] += _seg_sum(P, out * out)


def _center(z, bp, segz, cnt, gs):
    return pl.pallas_call(
        _center_body,
        grid=(NB,),
        in_specs=[_BS_ROWS, _BS_COL1, _full((G, D)), _full((1, G)),
                  _full((1, D))],
        out_specs=[_BS_ROWS, _BS_ACC_GD],
        out_shape=[jax.ShapeDtypeStruct((NPAD, D), jnp.float32),
                   jax.ShapeDtypeStruct((G, D), jnp.float32)],
    )(z, bp, segz, cnt, gs.reshape(1, D))


def _gate_body(last, out_ref, b_ref, segv_ref, cnt_ref, gwr, gbr, G1r, gb1r,
               G2r, gb2r, h_out, gate_out, mg_out):
    cnt = jnp.maximum(cnt_ref[...], 1.0).reshape(G, 1)
    inv = 1.0 / jnp.sqrt(segv_ref[...] / cnt + 1e-5)
    P = _onehot(b_ref)
    z = gwr[...] * out_ref[...] * _dot(P, inv, _HI) + gbr[...]
    if not last:
        z = jnp.maximum(z, 0.0)
    rows = lax.broadcasted_iota(jnp.int32, (R, 1), 0) + pl.program_id(0) * R
    z = jnp.where(rows < N, z, 0.0)
    h_out[...] = z
    gm = jnp.maximum(_dot(z, G1r[...]) + gb1r[...], 0.0)
    gate = _dot(gm, G2r[...]) + gb2r[...]
    gate_out[...] = gate

    @pl.when(pl.program_id(0) == 0)
    def _():
        mg_out[...] = jnp.full_like(mg_out, -jnp.inf)

    blockmax = jnp.max(jnp.where(P > 0, gate, -jnp.inf), axis=0, keepdims=True)
    mg_out[...] = jnp.maximum(mg_out[...], blockmax)


def _gate(last, out, bp, segv, cnt, gw, gb, G1, gb1, G2, gb2):
    return pl.pallas_call(
        functools.partial(_gate_body, last),
        grid=(NB,),
        in_specs=[_BS_ROWS, _BS_COL1, _full((G, D)), _full((1, G)),
                  _full((1, D)), _full((1, D)), _full((D, 2 * D)),
                  _full((1, 2 * D)), _full((2 * D, 1)), _full((1, 1))],
        out_specs=[_BS_ROWS, _BS_COL1, _BS_ACC_1G],
        out_shape=[jax.ShapeDtypeStruct((NPAD, D), jnp.float32),
                   jax.ShapeDtypeStruct((NPAD, 1), jnp.float32),
                   jax.ShapeDtypeStruct((1, G), jnp.float32)],
    )(out, bp, segv, cnt, gw.reshape(1, D), gb.reshape(1, D), G1,
      gb1.reshape(1, 2 * D), G2, gb2.reshape(1, 1))


def _pool_body(h_ref, gate_ref, b_ref, mg_ref, ssum_out, u_out):
    mg = mg_ref[...]
    mg = jnp.where(jnp.isfinite(mg), mg, 0.0)
    P = _onehot(b_ref)
    e = jnp.exp(gate_ref[...] - _dot(P, mg.reshape(G, 1), _HI))
    rows = lax.broadcasted_iota(jnp.int32, (R, 1), 0) + pl.program_id(0) * R
    e = jnp.where(rows < N, e, 0.0)

    @pl.when(pl.program_id(0) == 0)
    def _():
        ssum_out[...] = jnp.zeros_like(ssum_out)
        u_out[...] = jnp.zeros_like(u_out)

    ssum_out[...] += _seg_sum(P, e)
    u_out[...] += _seg_sum(P, e * h_ref[...])


def _pool(h, gate, bp, mg):
    return pl.pallas_call(
        _pool_body,
        grid=(NB,),
        in_specs=[_BS_ROWS, _BS_COL1, _BS_COL1, _full((1, G))],
        out_specs=[_BS_ACC_G1, _BS_ACC_GD],
        out_shape=[jax.ShapeDtypeStruct((G, 1), jnp.float32),
                   jax.ShapeDtypeStruct((G, D), jnp.float32)],
    )(h, gate, bp, mg)


def _final_body(u_ref, s_ref, out_ref):
    total = jnp.zeros((G, D), jnp.float32)
    for d in range(DEPTH):
        hg = u_ref[d] / jnp.maximum(s_ref[d], 1e-16)
        out_ref[d] = hg
        total = total + hg
    out_ref[DEPTH] = total * (1.0 / DEPTH)


def _final(us, ss):
    return pl.pallas_call(
        _final_body,
        out_shape=jax.ShapeDtypeStruct((DEPTH + 1, G, D), jnp.float32),
    )(us, ss)


def _layer(d, h, acc, bp, W1, b1, W2, b2, gw, gb, gs, G1, gb1, G2, gb2):
    last = d == DEPTH - 1
    z, segz, cnt = _mlp(h, acc, bp, W1, b1, W2, b2)
    out, segv = _center(z, bp, segz, cnt, gs)
    h_new, gate, mg = _gate(last, out, bp, segv, cnt, gw, gb, G1, gb1, G2, gb2)
    ssum, u = _pool(h_new, gate, bp, mg)
    return h_new, ssum, u


def kernel(x, edge_index, node_depth, batch, node_emb, depth_emb, W1, b1, W2, b2,
           gn_weight, gn_bias, gn_mean_scale, G1, gb1, G2, gb2):
    src = edge_index[0].astype(jnp.int32)
    dst = edge_index[1].astype(jnp.int32)
    src3 = jnp.concatenate([src, jnp.zeros((EPAD - E,), jnp.int32)]).reshape(NW, CH, K)
    dst3 = jnp.concatenate([dst, jnp.full((EPAD - E,), N, jnp.int32)]).reshape(NW, CH, K)
    xp = jnp.concatenate([x.astype(jnp.int32),
                          jnp.full((NPAD - N,), VOCAB, jnp.int32)]).reshape(NPAD, 1)
    dp = jnp.concatenate([node_depth.astype(jnp.int32),
                          jnp.full((NPAD - N,), MAXDEPTH, jnp.int32)]).reshape(NPAD, 1)
    bp = jnp.concatenate([batch.astype(jnp.int32),
                          jnp.full((NPAD - N,), G, jnp.int32)]).reshape(NPAD, 1)
    embp = jnp.zeros((128, D), jnp.float32).at[:VOCAB].set(node_emb)
    depthp = jnp.zeros((128, D), jnp.float32).at[:MAXDEPTH].set(depth_emb)

    h = _embed(xp, dp, embp, depthp)
    us, ss = [], []
    for d in range(DEPTH):
        acc = _sc_agg(h, src3, dst3)
        h, ssum, u = _layer(d, h, acc, bp, W1[d], b1[d], W2[d], b2[d],
                            gn_weight[d], gn_bias[d], gn_mean_scale[d],
                            G1[d], gb1[d], G2[d], gb2[d])
        us.append(u)
        ss.append(ssum)
    return _final(jnp.stack(us, axis=0), jnp.stack(ss, axis=0))
